# baseline (speedup 1.0000x reference)
import functools

import jax
import jax.numpy as jnp
from jax import lax
from jax.experimental import pallas as pl
from jax.experimental.pallas import tpu as pltpu
from jax.experimental.pallas import tpu_sc as plsc

N_NODES = 10000
D = 128
E = 320000
C = 64
NC, NS = 2, 16
NW = NC * NS
CPW = 160
ROWS_P = NW * CPW
E_P = ROWS_P * C
WB = 16
NB = CPW // WB
DUMMY_ROWS = 512
N_ACC = N_NODES + DUMMY_ROWS
SLICE = 656
TAIL = N_ACC - NS * SLICE
ZB = 16

_mesh = plsc.VectorSubcoreMesh(core_axis_name="c", subcore_axis_name="s")


def _fill_f32(ref, rows, cols, value):
    vec = jnp.full((16,), value, jnp.float32)

    @pl.loop(0, rows)
    def _(r):
        @pl.loop(0, cols // 16)
        def _(cc):
            ref.at[r, pl.ds(cc * 16, 16)][...] = vec


def _zero_spmem(acc_sh, zero_v, sid, width):
    base = sid * SLICE

    @pl.loop(0, SLICE // ZB)
    def _(k):
        pltpu.sync_copy(zero_v, acc_sh.at[pl.ds(base + k * ZB, ZB)])

    @pl.when(sid == NS - 1)
    def _():
        @pl.loop(0, TAIL // 8)
        def _(k):
            pltpu.sync_copy(zero_v.at[pl.ds(0, 8)],
                            acc_sh.at[pl.ds(NS * SLICE + k * 8, 8)])


def _writeback(acc_sh, acc_out, cid, sid):
    base = sid * SLICE
    pltpu.sync_copy(acc_sh.at[pl.ds(base, SLICE)],
                    acc_out.at[cid, pl.ds(base, SLICE)])

    @pl.when(sid == NS - 1)
    def _():
        pltpu.sync_copy(acc_sh.at[pl.ds(NS * SLICE, TAIL)],
                        acc_out.at[cid, pl.ds(NS * SLICE, TAIL)])


def _sc_agg_body(x_hbm, src_hbm, dst_hbm, acc_out,
                 acc_sh, src_b, dst_b, msg0, msg1, msg2, msg3, zero_v,
                 gsem0, gsem1, gsem2, gsem3, ssem0, ssem1, ssem2, ssem3):
    cid = lax.axis_index("c")
    sid = lax.axis_index("s")
    wid = cid * NS + sid

    _fill_f32(zero_v, ZB, D, 0.0)
    _zero_spmem(acc_sh, zero_v, sid, D)
    plsc.subcore_barrier()

    base_chunk = wid * CPW
    msgs = (msg0, msg1, msg2, msg3)
    gsems = (gsem0, gsem1, gsem2, gsem3)
    ssems = (ssem0, ssem1, ssem2, ssem3)

    @pl.loop(0, NB)
    def _(t):
        c0 = base_chunk + t * WB
        pltpu.sync_copy(src_hbm.at[pl.ds(c0, WB)], src_b)
        pltpu.sync_copy(dst_hbm.at[pl.ds(c0, WB)], dst_b)

        gd = [None] * WB
        sd = [None] * WB
        gd[0] = pltpu.async_copy(x_hbm.at[src_b.at[0]], msgs[0], gsems[0])
        gd[1] = pltpu.async_copy(x_hbm.at[src_b.at[1]], msgs[1], gsems[1])
        for k in range(WB):
            b = k % 4
            gd[k].wait()
            sd[k] = pltpu.async_copy(msgs[b], acc_sh.at[dst_b.at[k]],
                                     ssems[b], add=True)
            kk = k + 2
            if kk < WB:
                if kk >= 4:
                    sd[kk - 4].wait()
                gd[kk] = pltpu.async_copy(x_hbm.at[src_b.at[kk]],
                                          msgs[kk % 4], gsems[kk % 4])
        for j in range(WB - 4, WB):
            sd[j].wait()

    plsc.subcore_barrier()
    _writeback(acc_sh, acc_out, cid, sid)


def _sc_cnt_body(dst_hbm, cnt_out, cnt_sh, dst_b, ones_v, zero_v,
                 ssem0, ssem1):
    cid = lax.axis_index("c")
    sid = lax.axis_index("s")
    wid = cid * NS + sid

    _fill_f32(zero_v, ZB, D, 0.0)
    _fill_f32(ones_v, C, D, 1.0)
    _zero_spmem(cnt_sh, zero_v, sid, D)
    plsc.subcore_barrier()

    base_chunk = wid * CPW
    ssems = (ssem0, ssem1)

    @pl.loop(0, NB)
    def _(t):
        pltpu.sync_copy(dst_hbm.at[pl.ds(base_chunk + t * WB, WB)], dst_b)
        sd = [None, None]
        for j in range(WB):
            if sd[j % 2] is not None:
                sd[j % 2].wait()
            sd[j % 2] = pltpu.async_copy(ones_v, cnt_sh.at[dst_b.at[j]],
                                         ssems[j % 2], add=True)
        sd[0].wait()
        sd[1].wait()

    plsc.subcore_barrier()
    _writeback(cnt_sh, cnt_out, cid, sid)


_sc_agg = pl.kernel(
    _sc_agg_body,
    out_type=jax.ShapeDtypeStruct((NC, N_ACC, D), jnp.float32),
    mesh=_mesh,
    scratch_types=[
        pltpu.VMEM_SHARED((N_ACC, D), jnp.float32),
        pltpu.VMEM((WB, C), jnp.int32),
        pltpu.VMEM((WB, C), jnp.int32),
        pltpu.VMEM((C, D), jnp.float32),
        pltpu.VMEM((C, D), jnp.float32),
        pltpu.VMEM((C, D), jnp.float32),
        pltpu.VMEM((C, D), jnp.float32),
        pltpu.VMEM((ZB, D), jnp.float32),
    ] + [pltpu.SemaphoreType.DMA] * 8,
)

_sc_cnt = pl.kernel(
    _sc_cnt_body,
    out_type=jax.ShapeDtypeStruct((NC, N_ACC, D), jnp.float32),
    mesh=_mesh,
    scratch_types=[
        pltpu.VMEM_SHARED((N_ACC, D), jnp.float32),
        pltpu.VMEM((WB, C), jnp.int32),
        pltpu.VMEM((C, D), jnp.float32),
        pltpu.VMEM((ZB, D), jnp.float32),
        pltpu.SemaphoreType.DMA,
        pltpu.SemaphoreType.DMA,
    ],
)


def _combine_body(relu, acc_ref, cnt_ref, x_ref, wl_ref, wr_ref, b_ref, o_ref):
    acc = acc_ref[0] + acc_ref[1]
    cnt = cnt_ref[0, :, 0:1] + cnt_ref[1, :, 0:1]
    mean = acc / jnp.maximum(cnt, 1.0)
    y = (jnp.dot(mean, wl_ref[...], preferred_element_type=jnp.float32)
         + jnp.dot(x_ref[...], wr_ref[...], preferred_element_type=jnp.float32)
         + b_ref[...])
    o_ref[...] = jnp.maximum(y, 0.0) if relu else y


def _combine(acc, cnt, x, W_l, b_l, W_r, relu):
    blk = 1000
    return pl.pallas_call(
        functools.partial(_combine_body, relu),
        grid=(N_NODES // blk,),
        in_specs=[
            pl.BlockSpec((NC, blk, D), lambda j: (0, j, 0)),
            pl.BlockSpec((NC, blk, D), lambda j: (0, j, 0)),
            pl.BlockSpec((blk, D), lambda j: (j, 0)),
            pl.BlockSpec((D, D), lambda j: (0, 0)),
            pl.BlockSpec((D, D), lambda j: (0, 0)),
            pl.BlockSpec((1, D), lambda j: (0, 0)),
        ],
        out_specs=pl.BlockSpec((blk, D), lambda j: (j, 0)),
        out_shape=jax.ShapeDtypeStruct((N_NODES, D), jnp.float32),
    )(acc, cnt, x, W_l, W_r, b_l)


def kernel(x, edge_index, W1_l, b1_l, W1_r, W2_l, b2_l, W2_r):
    pad = E_P - E
    pad_src = (jnp.arange(pad, dtype=jnp.int32) * 131) % N_NODES
    pad_dst = N_NODES + (jnp.arange(pad, dtype=jnp.int32) % DUMMY_ROWS)
    src = jnp.concatenate([edge_index[0], pad_src]).reshape(ROWS_P, C)
    dst = jnp.concatenate([edge_index[1], pad_dst]).reshape(ROWS_P, C)
    b1 = b1_l.reshape(1, D)
    b2 = b2_l.reshape(1, D)

    cnt = _sc_cnt(dst)
    acc1 = _sc_agg(x, src, dst)
    h1 = _combine(acc1, cnt, x, W1_l, b1, W1_r, relu=True)
    acc2 = _sc_agg(h1, src, dst)
    return _combine(acc2, cnt, h1, W2_l, b2, W2_r, relu=False)


# speedup vs baseline: 1.1567x; 1.1567x over previous
import functools

import jax
import jax.numpy as jnp
from jax import lax
from jax.experimental import pallas as pl
from jax.experimental.pallas import tpu as pltpu
from jax.experimental.pallas import tpu_sc as plsc

N_NODES = 10000
D = 128
E = 320000
C = 128
NC, NS = 2, 16
NW = NC * NS
CPW = 80
ROWS_P = NW * CPW
E_P = ROWS_P * C
WB = 16
NB = CPW // WB
DUMMY_ROWS = 512
N_ACC = N_NODES + DUMMY_ROWS
SLICE = 656
TAIL = N_ACC - NS * SLICE
ZB = 16

_mesh = plsc.VectorSubcoreMesh(core_axis_name="c", subcore_axis_name="s")


def _fill_f32(ref, rows, cols, value):
    vec = jnp.full((16,), value, jnp.float32)

    @pl.loop(0, rows)
    def _(r):
        @pl.loop(0, cols // 16)
        def _(cc):
            ref.at[r, pl.ds(cc * 16, 16)][...] = vec


def _zero_spmem(acc_sh, zero_v, sid, width):
    base = sid * SLICE

    @pl.loop(0, SLICE // ZB)
    def _(k):
        pltpu.sync_copy(zero_v, acc_sh.at[pl.ds(base + k * ZB, ZB)])

    @pl.when(sid == NS - 1)
    def _():
        @pl.loop(0, TAIL // 8)
        def _(k):
            pltpu.sync_copy(zero_v.at[pl.ds(0, 8)],
                            acc_sh.at[pl.ds(NS * SLICE + k * 8, 8)])


def _writeback(acc_sh, acc_out, cid, sid):
    base = sid * SLICE
    pltpu.sync_copy(acc_sh.at[pl.ds(base, SLICE)],
                    acc_out.at[cid, pl.ds(base, SLICE)])

    @pl.when(sid == NS - 1)
    def _():
        pltpu.sync_copy(acc_sh.at[pl.ds(NS * SLICE, TAIL)],
                        acc_out.at[cid, pl.ds(NS * SLICE, TAIL)])


def _sc_agg_body(x_hbm, src_hbm, dst_hbm, acc_out,
                 acc_sh, src_b, dst_b, msg0, msg1, zero_v,
                 gsem0, gsem1, ssem0, ssem1, isem):
    cid = lax.axis_index("c")
    sid = lax.axis_index("s")
    wid = cid * NS + sid

    _fill_f32(zero_v, ZB, D, 0.0)
    _zero_spmem(acc_sh, zero_v, sid, D)
    plsc.subcore_barrier()

    base_chunk = wid * CPW
    msgs = (msg0, msg1)
    gsems = (gsem0, gsem1)
    ssems = (ssem0, ssem1)

    pltpu.sync_copy(src_hbm.at[pl.ds(base_chunk, WB)], src_b.at[0])
    pltpu.sync_copy(dst_hbm.at[pl.ds(base_chunk, WB)], dst_b.at[0])

    @pl.loop(0, NB)
    def _(t):
        cur = lax.rem(t, 2)
        nxt = lax.rem(t + 1, 2)

        @pl.when(t + 1 < NB)
        def _():
            c1 = base_chunk + (t + 1) * WB
            pltpu.async_copy(src_hbm.at[pl.ds(c1, WB)], src_b.at[nxt], isem)
            pltpu.async_copy(dst_hbm.at[pl.ds(c1, WB)], dst_b.at[nxt], isem)

        @pl.when(t > 0)
        def _():
            c0 = base_chunk + t * WB
            pltpu.make_async_copy(src_hbm.at[pl.ds(c0, WB)],
                                  src_b.at[cur], isem).wait()
            pltpu.make_async_copy(dst_hbm.at[pl.ds(c0, WB)],
                                  dst_b.at[cur], isem).wait()

        gd = [None, None]
        gd[0] = pltpu.async_copy(x_hbm.at[src_b.at[cur, 0]], msgs[0], gsems[0])
        gd[1] = pltpu.async_copy(x_hbm.at[src_b.at[cur, 1]], msgs[1], gsems[1])
        for j in range(WB):
            b = j % 2
            gd[b].wait()
            sd = pltpu.async_copy(msgs[b], acc_sh.at[dst_b.at[cur, j]],
                                  ssems[b], add=True)
            sd.wait()
            if j + 2 < WB:
                gd[b] = pltpu.async_copy(x_hbm.at[src_b.at[cur, j + 2]],
                                         msgs[b], gsems[b])

    plsc.subcore_barrier()
    _writeback(acc_sh, acc_out, cid, sid)


def _sc_cnt_body(dst_hbm, cnt_out, cnt_sh, dst_b, ones_v, zero_v,
                 ssem0, ssem1):
    cid = lax.axis_index("c")
    sid = lax.axis_index("s")
    wid = cid * NS + sid

    _fill_f32(zero_v, ZB, D, 0.0)
    _fill_f32(ones_v, C, D, 1.0)
    _zero_spmem(cnt_sh, zero_v, sid, D)
    plsc.subcore_barrier()

    base_chunk = wid * CPW
    ssems = (ssem0, ssem1)

    @pl.loop(0, NB)
    def _(t):
        pltpu.sync_copy(dst_hbm.at[pl.ds(base_chunk + t * WB, WB)], dst_b)
        sd = [None, None]
        for j in range(WB):
            if sd[j % 2] is not None:
                sd[j % 2].wait()
            sd[j % 2] = pltpu.async_copy(ones_v, cnt_sh.at[dst_b.at[j]],
                                         ssems[j % 2], add=True)
        sd[0].wait()
        sd[1].wait()

    plsc.subcore_barrier()
    _writeback(cnt_sh, cnt_out, cid, sid)


_sc_agg = pl.kernel(
    _sc_agg_body,
    out_type=jax.ShapeDtypeStruct((NC, N_ACC, D), jnp.float32),
    mesh=_mesh,
    scratch_types=[
        pltpu.VMEM_SHARED((N_ACC, D), jnp.float32),
        pltpu.VMEM((2, WB, C), jnp.int32),
        pltpu.VMEM((2, WB, C), jnp.int32),
        pltpu.VMEM((C, D), jnp.float32),
        pltpu.VMEM((C, D), jnp.float32),
        pltpu.VMEM((ZB, D), jnp.float32),
    ] + [pltpu.SemaphoreType.DMA] * 5,
)

_sc_cnt = pl.kernel(
    _sc_cnt_body,
    out_type=jax.ShapeDtypeStruct((NC, N_ACC, D), jnp.float32),
    mesh=_mesh,
    scratch_types=[
        pltpu.VMEM_SHARED((N_ACC, D), jnp.float32),
        pltpu.VMEM((WB, C), jnp.int32),
        pltpu.VMEM((C, D), jnp.float32),
        pltpu.VMEM((ZB, D), jnp.float32),
        pltpu.SemaphoreType.DMA,
        pltpu.SemaphoreType.DMA,
    ],
)


def _combine_body(relu, acc_ref, cnt_ref, x_ref, wl_ref, wr_ref, b_ref, o_ref):
    acc = acc_ref[0] + acc_ref[1]
    cnt = cnt_ref[0, :, 0:1] + cnt_ref[1, :, 0:1]
    mean = acc / jnp.maximum(cnt, 1.0)
    y = (jnp.dot(mean, wl_ref[...], preferred_element_type=jnp.float32)
         + jnp.dot(x_ref[...], wr_ref[...], preferred_element_type=jnp.float32)
         + b_ref[...])
    o_ref[...] = jnp.maximum(y, 0.0) if relu else y


def _combine(acc, cnt, x, W_l, b_l, W_r, relu):
    blk = 1000
    return pl.pallas_call(
        functools.partial(_combine_body, relu),
        grid=(N_NODES // blk,),
        in_specs=[
            pl.BlockSpec((NC, blk, D), lambda j: (0, j, 0)),
            pl.BlockSpec((NC, blk, D), lambda j: (0, j, 0)),
            pl.BlockSpec((blk, D), lambda j: (j, 0)),
            pl.BlockSpec((D, D), lambda j: (0, 0)),
            pl.BlockSpec((D, D), lambda j: (0, 0)),
            pl.BlockSpec((1, D), lambda j: (0, 0)),
        ],
        out_specs=pl.BlockSpec((blk, D), lambda j: (j, 0)),
        out_shape=jax.ShapeDtypeStruct((N_NODES, D), jnp.float32),
    )(acc, cnt, x, W_l, W_r, b_l)


def kernel(x, edge_index, W1_l, b1_l, W1_r, W2_l, b2_l, W2_r):
    pad = E_P - E
    pad_src = (jnp.arange(pad, dtype=jnp.int32) * 131) % N_NODES
    pad_dst = N_NODES + (jnp.arange(pad, dtype=jnp.int32) % DUMMY_ROWS)
    src = jnp.concatenate([edge_index[0], pad_src]).reshape(ROWS_P, C)
    dst = jnp.concatenate([edge_index[1], pad_dst]).reshape(ROWS_P, C)
    b1 = b1_l.reshape(1, D)
    b2 = b2_l.reshape(1, D)

    cnt = _sc_cnt(dst)
    acc1 = _sc_agg(x, src, dst)
    h1 = _combine(acc1, cnt, x, W1_l, b1, W1_r, relu=True)
    acc2 = _sc_agg(h1, src, dst)
    return _combine(acc2, cnt, h1, W2_l, b2, W2_r, relu=False)


# speedup vs baseline: 1.1944x; 1.0326x over previous
import functools

import jax
import jax.numpy as jnp
from jax import lax
from jax.experimental import pallas as pl
from jax.experimental.pallas import tpu as pltpu
from jax.experimental.pallas import tpu_sc as plsc

N_NODES = 10000
D = 128
E = 320000
C = 128
NC, NS = 2, 16
NW = NC * NS
CPW = 80
ROWS_P = NW * CPW
E_P = ROWS_P * C
WB = 16
NB = CPW // WB
DUMMY_ROWS = 512
N_ACC = N_NODES + DUMMY_ROWS
SLICE = 656
TAIL = N_ACC - NS * SLICE
ZB = 16

_mesh = plsc.VectorSubcoreMesh(core_axis_name="c", subcore_axis_name="s")


def _fill_f32(ref, rows, cols, value):
    vec = jnp.full((16,), value, jnp.float32)

    @pl.loop(0, rows)
    def _(r):
        @pl.loop(0, cols // 16)
        def _(cc):
            ref.at[r, pl.ds(cc * 16, 16)][...] = vec


def _zero_spmem(acc_sh, zero_v, sid, width):
    base = sid * SLICE

    @pl.loop(0, SLICE // ZB)
    def _(k):
        pltpu.sync_copy(zero_v, acc_sh.at[pl.ds(base + k * ZB, ZB)])

    @pl.when(sid == NS - 1)
    def _():
        @pl.loop(0, TAIL // 8)
        def _(k):
            pltpu.sync_copy(zero_v.at[pl.ds(0, 8)],
                            acc_sh.at[pl.ds(NS * SLICE + k * 8, 8)])


def _writeback(acc_sh, acc_out, cid, sid):
    base = sid * SLICE
    pltpu.sync_copy(acc_sh.at[pl.ds(base, SLICE)],
                    acc_out.at[cid, pl.ds(base, SLICE)])

    @pl.when(sid == NS - 1)
    def _():
        pltpu.sync_copy(acc_sh.at[pl.ds(NS * SLICE, TAIL)],
                        acc_out.at[cid, pl.ds(NS * SLICE, TAIL)])


def _sc_agg_body(x_hbm, src_hbm, dst_hbm, acc_out,
                 acc_sh, src_b, dst_b, msg0, msg1, zero_v,
                 gsem0, gsem1, ssem0, ssem1, isem):
    cid = lax.axis_index("c")
    sid = lax.axis_index("s")
    wid = cid * NS + sid

    _fill_f32(zero_v, ZB, D, 0.0)
    _zero_spmem(acc_sh, zero_v, sid, D)
    plsc.subcore_barrier()

    base_chunk = wid * CPW
    msgs = (msg0, msg1)
    gsems = (gsem0, gsem1)
    ssems = (ssem0, ssem1)

    pltpu.sync_copy(src_hbm.at[pl.ds(base_chunk, WB)], src_b.at[0])
    pltpu.sync_copy(dst_hbm.at[pl.ds(base_chunk, WB)], dst_b.at[0])

    @pl.loop(0, NB)
    def _(t):
        cur = lax.rem(t, 2)
        nxt = lax.rem(t + 1, 2)

        @pl.when(t + 1 < NB)
        def _():
            c1 = base_chunk + (t + 1) * WB
            pltpu.async_copy(src_hbm.at[pl.ds(c1, WB)], src_b.at[nxt], isem)
            pltpu.async_copy(dst_hbm.at[pl.ds(c1, WB)], dst_b.at[nxt], isem)

        @pl.when(t > 0)
        def _():
            c0 = base_chunk + t * WB
            pltpu.make_async_copy(src_hbm.at[pl.ds(c0, WB)],
                                  src_b.at[cur], isem).wait()
            pltpu.make_async_copy(dst_hbm.at[pl.ds(c0, WB)],
                                  dst_b.at[cur], isem).wait()

        gd = [None, None]
        gd[0] = pltpu.async_copy(x_hbm.at[src_b.at[cur, 0]], msgs[0], gsems[0])
        gd[1] = pltpu.async_copy(x_hbm.at[src_b.at[cur, 1]], msgs[1], gsems[1])
        for j in range(WB):
            b = j % 2
            gd[b].wait()
            sd = pltpu.async_copy(msgs[b], acc_sh.at[dst_b.at[cur, j]],
                                  ssems[b], add=True)
            sd.wait()
            if j + 2 < WB:
                gd[b] = pltpu.async_copy(x_hbm.at[src_b.at[cur, j + 2]],
                                         msgs[b], gsems[b])

    plsc.subcore_barrier()
    _writeback(acc_sh, acc_out, cid, sid)


def _sc_cnt_body(dst_hbm, cnt_out, cnt_sh, dst_b, ones_v, zero_v,
                 ssem0, ssem1):
    cid = lax.axis_index("c")
    sid = lax.axis_index("s")
    wid = cid * NS + sid

    _fill_f32(zero_v, ZB, D, 0.0)
    _fill_f32(ones_v, C, D, 1.0)
    _zero_spmem(cnt_sh, zero_v, sid, D)
    plsc.subcore_barrier()

    base_chunk = wid * CPW
    ssems = (ssem0, ssem1)

    @pl.loop(0, NB)
    def _(t):
        pltpu.sync_copy(dst_hbm.at[pl.ds(base_chunk + t * WB, WB)], dst_b)
        sd = [None, None]
        for j in range(WB):
            if sd[j % 2] is not None:
                sd[j % 2].wait()
            sd[j % 2] = pltpu.async_copy(ones_v, cnt_sh.at[dst_b.at[j]],
                                         ssems[j % 2], add=True)
        sd[0].wait()
        sd[1].wait()

    plsc.subcore_barrier()
    _writeback(cnt_sh, cnt_out, cid, sid)


_sc_agg = pl.kernel(
    _sc_agg_body,
    out_type=jax.ShapeDtypeStruct((NC, N_ACC, D), jnp.float32),
    mesh=_mesh,
    scratch_types=[
        pltpu.VMEM_SHARED((N_ACC, D), jnp.float32),
        pltpu.VMEM((2, WB, C), jnp.int32),
        pltpu.VMEM((2, WB, C), jnp.int32),
        pltpu.VMEM((C, D), jnp.float32),
        pltpu.VMEM((C, D), jnp.float32),
        pltpu.VMEM((ZB, D), jnp.float32),
    ] + [pltpu.SemaphoreType.DMA] * 5,
)

_sc_cnt = pl.kernel(
    _sc_cnt_body,
    out_type=jax.ShapeDtypeStruct((NC, N_ACC, D), jnp.float32),
    mesh=_mesh,
    scratch_types=[
        pltpu.VMEM_SHARED((N_ACC, D), jnp.float32),
        pltpu.VMEM((WB, C), jnp.int32),
        pltpu.VMEM((C, D), jnp.float32),
        pltpu.VMEM((ZB, D), jnp.float32),
        pltpu.SemaphoreType.DMA,
        pltpu.SemaphoreType.DMA,
    ],
)


def _prep_body(e_ref, src_ref, dst_ref):
    src = e_ref[0].reshape(E // C, C)
    dst = e_ref[1].reshape(E // C, C)
    idx = (jax.lax.broadcasted_iota(jnp.int32, (ROWS_P - E // C, C), 0) * C
           + jax.lax.broadcasted_iota(jnp.int32, (ROWS_P - E // C, C), 1))
    pad_src = (idx * 131) % N_NODES
    pad_dst = N_NODES + idx % DUMMY_ROWS
    src_ref[...] = jnp.concatenate([src, pad_src], axis=0)
    dst_ref[...] = jnp.concatenate([dst, pad_dst], axis=0)


def _prep(edge_index):
    out = jax.ShapeDtypeStruct((ROWS_P, C), jnp.int32)
    return pl.pallas_call(
        _prep_body,
        out_shape=(out, out),
    )(edge_index)


def _combine_body(relu, acc_ref, cnt_ref, x_ref, wl_ref, wr_ref, b_ref, o_ref):
    acc = acc_ref[0] + acc_ref[1]
    cnt = cnt_ref[0, :, 0:1] + cnt_ref[1, :, 0:1]
    mean = (acc / jnp.maximum(cnt, 1.0)).astype(jnp.bfloat16)
    y = (jnp.dot(mean, wl_ref[...].astype(jnp.bfloat16),
                 preferred_element_type=jnp.float32)
         + jnp.dot(x_ref[...].astype(jnp.bfloat16),
                   wr_ref[...].astype(jnp.bfloat16),
                   preferred_element_type=jnp.float32)
         + b_ref[...])
    o_ref[...] = jnp.maximum(y, 0.0) if relu else y


def _combine(acc, cnt, x, W_l, b_l, W_r, relu):
    blk = 1000
    return pl.pallas_call(
        functools.partial(_combine_body, relu),
        grid=(N_NODES // blk,),
        in_specs=[
            pl.BlockSpec((NC, blk, D), lambda j: (0, j, 0)),
            pl.BlockSpec((NC, blk, D), lambda j: (0, j, 0)),
            pl.BlockSpec((blk, D), lambda j: (j, 0)),
            pl.BlockSpec((D, D), lambda j: (0, 0)),
            pl.BlockSpec((D, D), lambda j: (0, 0)),
            pl.BlockSpec((1, D), lambda j: (0, 0)),
        ],
        out_specs=pl.BlockSpec((blk, D), lambda j: (j, 0)),
        out_shape=jax.ShapeDtypeStruct((N_NODES, D), jnp.float32),
    )(acc, cnt, x, W_l, W_r, b_l)


def kernel(x, edge_index, W1_l, b1_l, W1_r, W2_l, b2_l, W2_r):
    src, dst = _prep(edge_index)
    b1 = b1_l.reshape(1, D)
    b2 = b2_l.reshape(1, D)

    cnt = _sc_cnt(dst)
    acc1 = _sc_agg(x, src, dst)
    h1 = _combine(acc1, cnt, x, W1_l, b1, W1_r, relu=True)
    acc2 = _sc_agg(h1, src, dst)
    return _combine(acc2, cnt, h1, W2_l, b2, W2_r, relu=False)
